# baseline (speedup 1.0000x reference)
import functools

import jax
import jax.numpy as jnp
from jax import lax
from jax.experimental import pallas as pl
from jax.experimental.pallas import tpu as pltpu
from jax.experimental.pallas import tpu_sc as plsc

EMB = 64
NC, NS, L = 2, 16, 16
NW = NC * NS


def _sc_convert(t_gu, t_ga, t_mu, t_ma):
    V_U = t_gu.shape[1]
    V_A = t_ga.shape[1]
    BLK = 256
    mesh = plsc.VectorSubcoreMesh(core_axis_name="c", subcore_axis_name="s")

    @functools.partial(
        pl.kernel,
        out_type=[jax.ShapeDtypeStruct((V_U // 2, 2 * EMB), jnp.float32),
                  jax.ShapeDtypeStruct((V_A // 2, 2 * EMB), jnp.float32),
                  jax.ShapeDtypeStruct((V_U // 2, 2 * EMB), jnp.float32),
                  jax.ShapeDtypeStruct((V_A // 2, 2 * EMB), jnp.float32)],
        mesh=mesh,
        scratch_types=[
            pltpu.VMEM((EMB, BLK), jnp.float32),
            pltpu.VMEM((BLK // 2, 2 * EMB), jnp.float32),
            pltpu.SemaphoreType.DMA,
        ],
        compiler_params=pltpu.CompilerParams(needs_layout_passes=False),
    )
    def convert_kernel(gu, ga, mu, ma, o_gu, o_ga, o_mu, o_ma,
                       in0, tbuf, sem_in):
        cid = lax.axis_index("c")
        sid = lax.axis_index("s")

        lanes = lax.iota(jnp.int32, L)

        def transpose_cols(dst, col0, width):
            def trow(r, carry):
                p = lax.shift_right_logical(r, 1)
                half = lax.mul(lax.rem(r, 2), EMB)
                for c0 in range(0, EMB, L):
                    v = plsc.load_gather(in0, [c0 + lanes,
                                               jnp.broadcast_to(r, (L,))])
                    tbuf[p, pl.ds(half + c0, L)] = v
                return carry
            lax.fori_loop(0, width, trow, 0)
            p0 = pl.multiple_of(lax.div(col0, 2), EMB)
            pltpu.sync_copy(tbuf.at[pl.ds(0, width // 2)],
                            dst.at[pl.ds(p0, width // 2)])

        def do_table(src, dst):
            V = src.shape[1]
            nb = V // BLK
            nb_w = (nb + NS - 1) // NS

            def block_loop(i, carry):
                b = lax.min(sid * nb_w + i, nb - 1)
                col0 = pl.multiple_of(b * BLK, 2 * EMB)
                pltpu.async_copy(src.at[:, pl.ds(col0, BLK)], in0,
                                 sem_in).wait()
                transpose_cols(dst, col0, BLK)
                return carry
            lax.fori_loop(0, nb_w, block_loop, 0)

        @pl.when(cid == 0)
        def _():
            do_table(gu, o_gu)
            do_table(ga, o_ga)

        @pl.when(cid == 1)
        def _():
            do_table(mu, o_mu)
            do_table(ma, o_ma)

    return convert_kernel(t_gu, t_ga, t_mu, t_ma)


def _sc_gather_pairs(user_ids, artist_ids, pg_u, pg_a, pm_u, pm_a):
    B = user_ids.shape[0]
    b_per_w = B // NW
    CH = 128
    n_ch = b_per_w // CH
    mesh = plsc.VectorSubcoreMesh(core_axis_name="c", subcore_axis_name="s")

    @functools.partial(
        pl.kernel,
        out_type=[jax.ShapeDtypeStruct((B, 2 * EMB), jnp.float32)] * 4,
        mesh=mesh,
        scratch_types=[
            pltpu.VMEM((b_per_w,), jnp.int32),
            pltpu.VMEM((b_per_w,), jnp.int32),
            pltpu.VMEM((b_per_w,), jnp.int32),
            pltpu.VMEM((b_per_w,), jnp.int32),
            pltpu.VMEM((CH, 2 * EMB), jnp.float32),
            pltpu.VMEM((CH, 2 * EMB), jnp.float32),
            pltpu.SemaphoreType.DMA,
            pltpu.SemaphoreType.DMA,
        ],
    )
    def gather_kernel(uid, aid, gu, ga, mu, ma, o_gu, o_ga, o_mu, o_ma,
                      idx_u, idx_a, pidx_u, pidx_a, buf0, buf1, sem0, sem1):
        wid = lax.axis_index("s") * NC + lax.axis_index("c")
        base = wid * b_per_w

        pltpu.sync_copy(uid.at[pl.ds(base, b_per_w)], idx_u)
        pltpu.sync_copy(aid.at[pl.ds(base, b_per_w)], idx_a)
        for g in range(b_per_w // L):
            sl = pl.ds(g * L, L)
            pidx_u[sl] = lax.shift_right_logical(idx_u[sl], 1)
            pidx_a[sl] = lax.shift_right_logical(idx_a[sl], 1)

        def fire(task, buf):
            table, pidx, _, c = task
            rows, sem = buf
            pltpu.async_copy(table.at[pidx.at[pl.ds(c * CH, CH)]], rows, sem)

        def finish(task, buf):
            table, _, out, c = task
            rows, sem = buf
            pltpu.make_async_copy(table.at[pl.ds(0, CH)], rows, sem).wait()
            pltpu.sync_copy(rows, out.at[pl.ds(base + c * CH, CH)])

        tasks = [(t, piv, o, c)
                 for (t, piv, o) in ((gu, pidx_u, o_gu), (ga, pidx_a, o_ga),
                                     (mu, pidx_u, o_mu), (ma, pidx_a, o_ma))
                 for c in range(n_ch)]
        bufs = [(buf0, sem0), (buf1, sem1)]
        for k, task in enumerate(tasks):
            if k >= 2:
                finish(tasks[k - 2], bufs[k % 2])
            fire(task, bufs[k % 2])
        finish(tasks[-2], bufs[len(tasks) % 2])
        finish(tasks[-1], bufs[(len(tasks) + 1) % 2])

    return gather_kernel(user_ids, artist_ids, pg_u, pg_a, pm_u, pm_a)


def _tc_mlp(pr_gu, pr_ga, pr_mu, pr_ma, uid_c, aid_c, ts_u, ts_a,
            tl_gu, tl_ga, tl_mu, tl_ma, W1, b1, W2, b2, W3, b3, Wf, bf):
    B = pr_gu.shape[0]
    BB = 2048
    TW_U = tl_gu.shape[1]
    TW_A = tl_ga.shape[1]
    TS_U = ts_u
    TS_A = ts_a
    w1u = W1[:, :EMB].T
    w1a = W1[:, EMB:].T
    w2t = W2.T
    w3t = W3.T
    wfg = Wf[:, :EMB]
    wfh = Wf[:, EMB:]
    b1r = b1.reshape(1, -1)
    b2r = b2.reshape(1, -1)
    b3r = b3.reshape(1, -1)
    bfr = bf.reshape(1, 1)

    def body(pgu, pga, pmu, pma, uu, aa, tgu, tga, tmu, tma,
             w1u_r, w1a_r, w2_r, w3_r, wfg_r, wfh_r,
             b1_r, b2_r, b3_r, bf_r, out_r):
        uid = uu[...]
        aid = aa[...]
        su = (uid & 1) == 1
        sa = (aid & 1) == 1
        dg = functools.partial(lax.dot_general,
                               dimension_numbers=(((1,), (1,)), ((), ())),
                               preferred_element_type=jnp.float32)
        tu = (uid >= TS_U)
        ta = (aid >= TS_A)
        oh_u = jnp.where(uid == TS_U + jax.lax.broadcasted_iota(
            jnp.int32, (1, TW_U), 1), 1.0, 0.0)
        oh_a = jnp.where(aid == TS_A + jax.lax.broadcasted_iota(
            jnp.int32, (1, TW_A), 1), 1.0, 0.0)

        def pick(pair, sel, intail, oh, tl):
            m = jnp.where(sel, pair[:, EMB:], pair[:, :EMB])
            return jnp.where(intail, dg(oh, tl[...]), m)

        gu = pick(pgu[...], su, tu, oh_u, tgu)
        ga = pick(pga[...], sa, ta, oh_a, tga)
        mu = pick(pmu[...], su, tu, oh_u, tmu)
        ma = pick(pma[...], sa, ta, oh_a, tma)
        dot = functools.partial(jnp.dot, preferred_element_type=jnp.float32)
        h = jnp.maximum(dot(mu, w1u_r[...]) + dot(ma, w1a_r[...])
                        + b1_r[...], 0.0)
        h = jnp.maximum(dot(h, w2_r[...]) + b2_r[...], 0.0)
        h = jnp.maximum(dot(h, w3_r[...]) + b3_r[...], 0.0)
        g = jnp.sum(gu * ga * wfg_r[...], axis=1, keepdims=True)
        m = jnp.sum(h * wfh_r[...], axis=1, keepdims=True)
        out_r[...] = jax.nn.sigmoid(g + m + bf_r[...])

    full = lambda a: pl.BlockSpec(a.shape, lambda i: (0, 0))
    pblk = pl.BlockSpec((BB, 2 * EMB), lambda i: (i, 0))
    iblk = pl.BlockSpec((BB, 1), lambda i: (i, 0))
    out = pl.pallas_call(
        body,
        grid=(B // BB,),
        in_specs=[pblk, pblk, pblk, pblk, iblk, iblk,
                  full(tl_gu), full(tl_ga), full(tl_mu), full(tl_ma),
                  full(w1u), full(w1a), full(w2t), full(w3t),
                  full(wfg), full(wfh), full(b1r), full(b2r), full(b3r),
                  full(bfr)],
        out_specs=pl.BlockSpec((BB, 1), lambda i: (i, 0)),
        out_shape=jax.ShapeDtypeStruct((B, 1), jnp.float32),
    )(pr_gu, pr_ga, pr_mu, pr_ma, uid_c, aid_c, tl_gu, tl_ga, tl_mu, tl_ma,
      w1u, w1a, w2t, w3t, wfg, wfh, b1r, b2r, b3r, bfr)
    return out[:, 0]


def kernel(user_ids, artist_ids, gmf_user, gmf_artist, mlp_user, mlp_artist,
           W1, b1, W2, b2, W3, b3, Wf, bf):
    t_gu, t_ga = gmf_user.T, gmf_artist.T
    t_mu, t_ma = mlp_user.T, mlp_artist.T
    pg_u, pg_a, pm_u, pm_a = _sc_convert(t_gu, t_ga, t_mu, t_ma)
    pr_gu, pr_ga, pr_mu, pr_ma = _sc_gather_pairs(
        user_ids, artist_ids, pg_u, pg_a, pm_u, pm_a)
    V_U = gmf_user.shape[0]
    V_A = gmf_artist.shape[0]
    BLK = 256
    ts_u, ts_a = (V_U // BLK) * BLK, (V_A // BLK) * BLK
    tl_gu = lax.slice(t_gu, (0, ts_u), (EMB, V_U))
    tl_mu = lax.slice(t_mu, (0, ts_u), (EMB, V_U))
    tl_ga = lax.slice(t_ga, (0, ts_a), (EMB, V_A))
    tl_ma = lax.slice(t_ma, (0, ts_a), (EMB, V_A))
    uid_c = user_ids.reshape(-1, 1)
    aid_c = artist_ids.reshape(-1, 1)
    return _tc_mlp(pr_gu, pr_ga, pr_mu, pr_ma, uid_c, aid_c, ts_u, ts_a,
                   tl_gu, tl_ga, tl_mu, tl_ma,
                   W1, b1, W2, b2, W3, b3, Wf, bf)


# speedup vs baseline: 1.1866x; 1.1866x over previous
import functools

import jax
import jax.numpy as jnp
from jax import lax
from jax.experimental import pallas as pl
from jax.experimental.pallas import tpu as pltpu
from jax.experimental.pallas import tpu_sc as plsc

EMB = 64
NC, NS, L = 2, 16, 16
NW = NC * NS


def _sc_convert(t_gu, t_ga, t_mu, t_ma):
    V_U = t_gu.shape[1]
    V_A = t_ga.shape[1]
    BLK = 256
    mesh = plsc.VectorSubcoreMesh(core_axis_name="c", subcore_axis_name="s")

    @functools.partial(
        pl.kernel,
        out_type=[jax.ShapeDtypeStruct((V_U // 2, 2 * EMB), jnp.float32),
                  jax.ShapeDtypeStruct((V_A // 2, 2 * EMB), jnp.float32),
                  jax.ShapeDtypeStruct((V_U // 2, 2 * EMB), jnp.float32),
                  jax.ShapeDtypeStruct((V_A // 2, 2 * EMB), jnp.float32)],
        mesh=mesh,
        scratch_types=[
            pltpu.VMEM((EMB, BLK), jnp.float32),
            pltpu.VMEM((BLK // 2, 2 * EMB), jnp.float32),
            pltpu.SemaphoreType.DMA,
        ],
        compiler_params=pltpu.CompilerParams(needs_layout_passes=False),
    )
    def convert_kernel(gu, ga, mu, ma, o_gu, o_ga, o_mu, o_ma,
                       in0, tbuf, sem_in):
        cid = lax.axis_index("c")
        sid = lax.axis_index("s")

        lanes = lax.iota(jnp.int32, L)

        def transpose_cols(dst, col0, width):
            n_g = width // L

            def titer(it, carry):
                c = lax.div(it, n_g)
                g = lax.rem(it, n_g)
                r = lanes + g * L
                rows = lax.shift_right_logical(r, 1)
                cols = lax.mul(lax.rem(r, 2), EMB) + c
                v = plsc.load_gather(in0, [jnp.broadcast_to(c, (L,)), r])
                plsc.store_scatter(tbuf, [rows, cols], v)
                return carry
            lax.fori_loop(0, EMB * n_g, titer, 0)
            p0 = pl.multiple_of(lax.div(col0, 2), EMB)
            pltpu.sync_copy(tbuf.at[pl.ds(0, width // 2)],
                            dst.at[pl.ds(p0, width // 2)])

        def do_table(src, dst):
            V = src.shape[1]
            nb = V // BLK
            nb_w = (nb + NS - 1) // NS

            def block_loop(i, carry):
                b = lax.min(sid * nb_w + i, nb - 1)
                col0 = pl.multiple_of(b * BLK, 2 * EMB)
                pltpu.async_copy(src.at[:, pl.ds(col0, BLK)], in0,
                                 sem_in).wait()
                transpose_cols(dst, col0, BLK)
                return carry
            lax.fori_loop(0, nb_w, block_loop, 0)

        @pl.when(cid == 0)
        def _():
            do_table(gu, o_gu)
            do_table(ga, o_ga)

        @pl.when(cid == 1)
        def _():
            do_table(mu, o_mu)
            do_table(ma, o_ma)

    return convert_kernel(t_gu, t_ga, t_mu, t_ma)


def _sc_gather_pairs(user_ids, artist_ids, pg_u, pg_a, pm_u, pm_a):
    B = user_ids.shape[0]
    b_per_w = B // NW
    CH = 128
    n_ch = b_per_w // CH
    mesh = plsc.VectorSubcoreMesh(core_axis_name="c", subcore_axis_name="s")

    @functools.partial(
        pl.kernel,
        out_type=[jax.ShapeDtypeStruct((B, 2 * EMB), jnp.float32)] * 4,
        mesh=mesh,
        scratch_types=[
            pltpu.VMEM((b_per_w,), jnp.int32),
            pltpu.VMEM((b_per_w,), jnp.int32),
            pltpu.VMEM((b_per_w,), jnp.int32),
            pltpu.VMEM((b_per_w,), jnp.int32),
            pltpu.VMEM((CH, 2 * EMB), jnp.float32),
            pltpu.VMEM((CH, 2 * EMB), jnp.float32),
            pltpu.SemaphoreType.DMA,
            pltpu.SemaphoreType.DMA,
        ],
    )
    def gather_kernel(uid, aid, gu, ga, mu, ma, o_gu, o_ga, o_mu, o_ma,
                      idx_u, idx_a, pidx_u, pidx_a, buf0, buf1, sem0, sem1):
        wid = lax.axis_index("s") * NC + lax.axis_index("c")
        base = wid * b_per_w

        pltpu.sync_copy(uid.at[pl.ds(base, b_per_w)], idx_u)
        pltpu.sync_copy(aid.at[pl.ds(base, b_per_w)], idx_a)
        for g in range(b_per_w // L):
            sl = pl.ds(g * L, L)
            pidx_u[sl] = lax.shift_right_logical(idx_u[sl], 1)
            pidx_a[sl] = lax.shift_right_logical(idx_a[sl], 1)

        def fire(task, buf):
            table, pidx, _, c = task
            rows, sem = buf
            pltpu.async_copy(table.at[pidx.at[pl.ds(c * CH, CH)]], rows, sem)

        def finish(task, buf):
            table, _, out, c = task
            rows, sem = buf
            pltpu.make_async_copy(table.at[pl.ds(0, CH)], rows, sem).wait()
            pltpu.sync_copy(rows, out.at[pl.ds(base + c * CH, CH)])

        tasks = [(t, piv, o, c)
                 for (t, piv, o) in ((gu, pidx_u, o_gu), (ga, pidx_a, o_ga),
                                     (mu, pidx_u, o_mu), (ma, pidx_a, o_ma))
                 for c in range(n_ch)]
        bufs = [(buf0, sem0), (buf1, sem1)]
        for k, task in enumerate(tasks):
            if k >= 2:
                finish(tasks[k - 2], bufs[k % 2])
            fire(task, bufs[k % 2])
        finish(tasks[-2], bufs[len(tasks) % 2])
        finish(tasks[-1], bufs[(len(tasks) + 1) % 2])

    return gather_kernel(user_ids, artist_ids, pg_u, pg_a, pm_u, pm_a)


def _tc_mlp(pr_gu, pr_ga, pr_mu, pr_ma, uid_c, aid_c, ts_u, ts_a,
            tl_gu, tl_ga, tl_mu, tl_ma, W1, b1, W2, b2, W3, b3, Wf, bf):
    B = pr_gu.shape[0]
    BB = 2048
    TW_U = tl_gu.shape[1]
    TW_A = tl_ga.shape[1]
    TS_U = ts_u
    TS_A = ts_a
    w1u = W1[:, :EMB].T
    w1a = W1[:, EMB:].T
    w2t = W2.T
    w3t = W3.T
    wfg = Wf[:, :EMB]
    wfh = Wf[:, EMB:]
    b1r = b1.reshape(1, -1)
    b2r = b2.reshape(1, -1)
    b3r = b3.reshape(1, -1)
    bfr = bf.reshape(1, 1)

    def body(pgu, pga, pmu, pma, uu, aa, tgu, tga, tmu, tma,
             w1u_r, w1a_r, w2_r, w3_r, wfg_r, wfh_r,
             b1_r, b2_r, b3_r, bf_r, out_r):
        uid = uu[...]
        aid = aa[...]
        su = (uid & 1) == 1
        sa = (aid & 1) == 1
        dg = functools.partial(lax.dot_general,
                               dimension_numbers=(((1,), (1,)), ((), ())),
                               preferred_element_type=jnp.float32)
        tu = (uid >= TS_U)
        ta = (aid >= TS_A)
        oh_u = jnp.where(uid == TS_U + jax.lax.broadcasted_iota(
            jnp.int32, (1, TW_U), 1), 1.0, 0.0)
        oh_a = jnp.where(aid == TS_A + jax.lax.broadcasted_iota(
            jnp.int32, (1, TW_A), 1), 1.0, 0.0)

        def pick(pair, sel, intail, oh, tl):
            m = jnp.where(sel, pair[:, EMB:], pair[:, :EMB])
            return jnp.where(intail, dg(oh, tl[...]), m)

        gu = pick(pgu[...], su, tu, oh_u, tgu)
        ga = pick(pga[...], sa, ta, oh_a, tga)
        mu = pick(pmu[...], su, tu, oh_u, tmu)
        ma = pick(pma[...], sa, ta, oh_a, tma)
        dot = functools.partial(jnp.dot, preferred_element_type=jnp.float32)
        h = jnp.maximum(dot(mu, w1u_r[...]) + dot(ma, w1a_r[...])
                        + b1_r[...], 0.0)
        h = jnp.maximum(dot(h, w2_r[...]) + b2_r[...], 0.0)
        h = jnp.maximum(dot(h, w3_r[...]) + b3_r[...], 0.0)
        g = jnp.sum(gu * ga * wfg_r[...], axis=1, keepdims=True)
        m = jnp.sum(h * wfh_r[...], axis=1, keepdims=True)
        out_r[...] = jax.nn.sigmoid(g + m + bf_r[...])

    full = lambda a: pl.BlockSpec(a.shape, lambda i: (0, 0))
    pblk = pl.BlockSpec((BB, 2 * EMB), lambda i: (i, 0))
    iblk = pl.BlockSpec((BB, 1), lambda i: (i, 0))
    out = pl.pallas_call(
        body,
        grid=(B // BB,),
        in_specs=[pblk, pblk, pblk, pblk, iblk, iblk,
                  full(tl_gu), full(tl_ga), full(tl_mu), full(tl_ma),
                  full(w1u), full(w1a), full(w2t), full(w3t),
                  full(wfg), full(wfh), full(b1r), full(b2r), full(b3r),
                  full(bfr)],
        out_specs=pl.BlockSpec((BB, 1), lambda i: (i, 0)),
        out_shape=jax.ShapeDtypeStruct((B, 1), jnp.float32),
    )(pr_gu, pr_ga, pr_mu, pr_ma, uid_c, aid_c, tl_gu, tl_ga, tl_mu, tl_ma,
      w1u, w1a, w2t, w3t, wfg, wfh, b1r, b2r, b3r, bfr)
    return out[:, 0]


def kernel(user_ids, artist_ids, gmf_user, gmf_artist, mlp_user, mlp_artist,
           W1, b1, W2, b2, W3, b3, Wf, bf):
    t_gu, t_ga = gmf_user.T, gmf_artist.T
    t_mu, t_ma = mlp_user.T, mlp_artist.T
    pg_u, pg_a, pm_u, pm_a = _sc_convert(t_gu, t_ga, t_mu, t_ma)
    pr_gu, pr_ga, pr_mu, pr_ma = _sc_gather_pairs(
        user_ids, artist_ids, pg_u, pg_a, pm_u, pm_a)
    V_U = gmf_user.shape[0]
    V_A = gmf_artist.shape[0]
    BLK = 256
    ts_u, ts_a = (V_U // BLK) * BLK, (V_A // BLK) * BLK
    tl_gu = lax.slice(t_gu, (0, ts_u), (EMB, V_U))
    tl_mu = lax.slice(t_mu, (0, ts_u), (EMB, V_U))
    tl_ga = lax.slice(t_ga, (0, ts_a), (EMB, V_A))
    tl_ma = lax.slice(t_ma, (0, ts_a), (EMB, V_A))
    uid_c = user_ids.reshape(-1, 1)
    aid_c = artist_ids.reshape(-1, 1)
    return _tc_mlp(pr_gu, pr_ga, pr_mu, pr_ma, uid_c, aid_c, ts_u, ts_a,
                   tl_gu, tl_ga, tl_mu, tl_ma,
                   W1, b1, W2, b2, W3, b3, Wf, bf)


# speedup vs baseline: 4.8938x; 4.1243x over previous
import functools

import jax
import jax.numpy as jnp
from jax import lax
from jax.experimental import pallas as pl
from jax.experimental.pallas import tpu as pltpu
from jax.experimental.pallas import tpu_sc as plsc

EMB = 64
NC, NS, L = 2, 16, 16
NW = NC * NS


def _sc_gather4(user_ids, artist_ids, gmf_user, gmf_artist, mlp_user, mlp_artist):
    B = user_ids.shape[0]
    b_per_w = B // NW
    CH = b_per_w // 2
    mesh = plsc.VectorSubcoreMesh(core_axis_name="c", subcore_axis_name="s")

    @functools.partial(
        pl.kernel,
        out_type=[jax.ShapeDtypeStruct((B, EMB), jnp.float32)] * 4,
        mesh=mesh,
        scratch_types=[
            pltpu.VMEM((b_per_w,), jnp.int32),
            pltpu.VMEM((b_per_w,), jnp.int32),
            pltpu.VMEM((CH, EMB), jnp.float32),
            pltpu.VMEM((CH, EMB), jnp.float32),
            pltpu.SemaphoreType.DMA,
            pltpu.SemaphoreType.DMA,
        ],
    )
    def gather_kernel(uid, aid, gu, ga, mu, ma, o_gu, o_ga, o_mu, o_ma,
                      idx_u, idx_a, rows0, rows1, sem0, sem1):
        wid = lax.axis_index("s") * NC + lax.axis_index("c")
        base = wid * b_per_w

        def fire(task, buf):
            table, idx_v, _, chunk = task
            rows, sem = buf

            def lbody(g, carry):
                vec = idx_v[pl.ds(chunk * CH + g * L, L)]
                for k in range(L):
                    pltpu.async_copy(table.at[pl.ds(vec[k], 1)],
                                     rows.at[pl.ds(g * L + k, 1)], sem)
                return carry
            lax.fori_loop(0, CH // L, lbody, 0)

        def finish(task, buf):
            table, _, out, chunk = task
            rows, sem = buf
            pltpu.make_async_copy(table.at[pl.ds(0, CH)], rows, sem).wait()
            pltpu.sync_copy(rows, out.at[pl.ds(base + chunk * CH, CH)])

        pltpu.sync_copy(uid.at[pl.ds(base, b_per_w)], idx_u)
        pltpu.sync_copy(aid.at[pl.ds(base, b_per_w)], idx_a)

        tasks = [(t, iv, o, c)
                 for (t, iv, o) in ((gu, idx_u, o_gu), (ga, idx_a, o_ga),
                                    (mu, idx_u, o_mu), (ma, idx_a, o_ma))
                 for c in (0, 1)]
        bufs = [(rows0, sem0), (rows1, sem1)]
        for k, task in enumerate(tasks):
            if k >= 2:
                finish(tasks[k - 2], bufs[k % 2])
            fire(task, bufs[k % 2])
        finish(tasks[-2], bufs[0])
        finish(tasks[-1], bufs[1])

    return gather_kernel(user_ids, artist_ids, gmf_user, gmf_artist,
                         mlp_user, mlp_artist)


def _tc_mlp(gmf_u, gmf_a, mlp_u, mlp_a, W1, b1, W2, b2, W3, b3, Wf, bf):
    B = gmf_u.shape[0]
    BB = 2048
    w1u = W1[:, :EMB].T
    w1a = W1[:, EMB:].T
    w2t = W2.T
    w3t = W3.T
    wfg = Wf[:, :EMB]
    wfh = Wf[:, EMB:]
    b1r = b1.reshape(1, -1)
    b2r = b2.reshape(1, -1)
    b3r = b3.reshape(1, -1)
    bfr = bf.reshape(1, 1)

    def body(gu, ga, mu, ma, w1u_r, w1a_r, w2_r, w3_r, wfg_r, wfh_r,
             b1_r, b2_r, b3_r, bf_r, out_r):
        dot = functools.partial(jnp.dot, preferred_element_type=jnp.float32)
        h = jnp.maximum(dot(mu[...], w1u_r[...]) + dot(ma[...], w1a_r[...])
                        + b1_r[...], 0.0)
        h = jnp.maximum(dot(h, w2_r[...]) + b2_r[...], 0.0)
        h = jnp.maximum(dot(h, w3_r[...]) + b3_r[...], 0.0)
        g = jnp.sum(gu[...] * ga[...] * wfg_r[...], axis=1, keepdims=True)
        m = jnp.sum(h * wfh_r[...], axis=1, keepdims=True)
        out_r[...] = jax.nn.sigmoid(g + m + bf_r[...])

    full = lambda a: pl.BlockSpec(a.shape, lambda i: (0, 0))
    blk = pl.BlockSpec((BB, EMB), lambda i: (i, 0))
    out = pl.pallas_call(
        body,
        grid=(B // BB,),
        in_specs=[blk, blk, blk, blk,
                  full(w1u), full(w1a), full(w2t), full(w3t),
                  full(wfg), full(wfh), full(b1r), full(b2r), full(b3r),
                  full(bfr)],
        out_specs=pl.BlockSpec((BB, 1), lambda i: (i, 0)),
        out_shape=jax.ShapeDtypeStruct((B, 1), jnp.float32),
    )(gmf_u, gmf_a, mlp_u, mlp_a, w1u, w1a, w2t, w3t, wfg, wfh,
      b1r, b2r, b3r, bfr)
    return out[:, 0]


def kernel(user_ids, artist_ids, gmf_user, gmf_artist, mlp_user, mlp_artist,
           W1, b1, W2, b2, W3, b3, Wf, bf):
    gu, ga, mu, ma = _sc_gather4(user_ids, artist_ids, gmf_user, gmf_artist,
                                 mlp_user, mlp_artist)
    return _tc_mlp(gu, ga, mu, ma, W1, b1, W2, b2, W3, b3, Wf, bf)
